# baseline (device time: 107031 ns/iter reference)
import jax
import jax.numpy as jnp
from jax import lax
from jax.experimental import pallas as pl
from jax.experimental.pallas import tpu as pltpu

N_DEV = 32
SQ = 1024
D = 1024
HQ = 256
HQ_LOCAL = 8
DH = 128
BLK = 64
SCALE = 0.08838834764831843
CHUNK = SQ // N_DEV
NBLK = 4
RB = SQ // NBLK
CPB = RB // CHUNK


def _rs_desc(pbuf, rs_buf, rs_send, rs_recv, my, c):
    return pltpu.make_async_remote_copy(
        src_ref=pbuf.at[pl.ds(c * CHUNK, CHUNK), :],
        dst_ref=rs_buf.at[my],
        send_sem=rs_send.at[c],
        recv_sem=rs_recv.at[my],
        device_id=(c,),
        device_id_type=pl.DeviceIdType.MESH,
    )


def _fused_body(x_ref, wq_ref, k_any, v_any, wo_ref, out_ref,
                k_vmem, v_vmem, pbuf, rs_buf, ag_buf,
                kv_sems, rs_send, rs_recv, ag_send, ag_recv):
    my = lax.axis_index("i")
    h0 = my * HQ_LOCAL

    kv_copies = []
    for h in range(HQ_LOCAL):
        ck = pltpu.make_async_copy(
            k_any.at[:, h0 + h, :], k_vmem.at[h], kv_sems.at[h])
        cv = pltpu.make_async_copy(
            v_any.at[:, h0 + h, :], v_vmem.at[h], kv_sems.at[HQ_LOCAL + h])
        ck.start()
        cv.start()
        kv_copies.append(ck)
        kv_copies.append(cv)

    Q = jnp.dot(x_ref[:, :].astype(jnp.bfloat16),
                wq_ref[:, :].astype(jnp.bfloat16),
                preferred_element_type=jnp.float32)
    Qs = (Q * SCALE).astype(jnp.bfloat16)
    Wo = wo_ref[:, :].astype(jnp.bfloat16)

    for c in kv_copies:
        c.wait()

    k16 = [k_vmem[h].astype(jnp.bfloat16) for h in range(HQ_LOCAL)]
    v16 = [v_vmem[h].astype(jnp.bfloat16) for h in range(HQ_LOCAL)]

    for b in range(NBLK):
        r0 = b * RB
        ncol = (b + 1) * RB
        qblk = (r0 + lax.broadcasted_iota(jnp.int32, (RB, ncol), 0)) // BLK
        kblk = lax.broadcasted_iota(jnp.int32, (RB, ncol), 1) // BLK
        mask = kblk <= qblk
        ctx_parts = []
        for h in range(HQ_LOCAL):
            q_h = Qs[r0:r0 + RB, h * DH:(h + 1) * DH]
            s = lax.dot_general(
                q_h, k16[h][:ncol, :], (((1,), (1,)), ((), ())),
                preferred_element_type=jnp.float32,
            )
            s = jnp.where(mask, s, -1e9)
            m = jnp.max(s, axis=1, keepdims=True)
            w = jnp.exp(s - m)
            w = w / jnp.sum(w, axis=1, keepdims=True)
            ctx_parts.append(
                jnp.dot(w.astype(jnp.bfloat16), v16[h][:ncol, :],
                        preferred_element_type=jnp.float32))
        ctx_b = jnp.concatenate(ctx_parts, axis=1)
        pb = jnp.dot(ctx_b.astype(jnp.bfloat16), Wo,
                     preferred_element_type=jnp.float32)
        pbuf[pl.ds(r0, RB), :] = pb.astype(jnp.bfloat16)

        for j in range(CPB):
            c = b * CPB + j

            @pl.when(my != c)
            def _(c=c):
                _rs_desc(pbuf, rs_buf, rs_send, rs_recv, my, c).start()

    pbuf_my = pbuf[pl.ds(my * CHUNK, CHUNK), :]
    rs_buf[pl.ds(my, 1)] = pbuf_my[None, :, :]

    for k in range(1, N_DEV):
        s = lax.rem(my + k, N_DEV)
        pltpu.make_async_remote_copy(
            src_ref=pbuf.at[pl.ds(0, CHUNK), :],
            dst_ref=rs_buf.at[s],
            send_sem=rs_send.at[0],
            recv_sem=rs_recv.at[s],
            device_id=(s,),
            device_id_type=pl.DeviceIdType.MESH,
        ).wait_recv()

    red = jnp.sum(rs_buf[:, :, :].astype(jnp.float32), axis=0)
    ag_buf[pl.ds(my, 1)] = red.astype(jnp.bfloat16)[None, :, :]

    ag_descs = []
    for k in range(1, N_DEV):
        peer = lax.rem(my + k, N_DEV)
        rdma = pltpu.make_async_remote_copy(
            src_ref=ag_buf.at[my],
            dst_ref=ag_buf.at[my],
            send_sem=ag_send.at[k - 1],
            recv_sem=ag_recv.at[my],
            device_id=(peer,),
            device_id_type=pl.DeviceIdType.MESH,
        )
        rdma.start()
        ag_descs.append(rdma)

    for k in range(1, N_DEV):
        s = lax.rem(my + k, N_DEV)
        pltpu.make_async_remote_copy(
            src_ref=ag_buf.at[0],
            dst_ref=ag_buf.at[s],
            send_sem=ag_send.at[0],
            recv_sem=ag_recv.at[s],
            device_id=(s,),
            device_id_type=pl.DeviceIdType.MESH,
        ).wait_recv()

    out_ref[:, :] = ag_buf[:, :, :].reshape(SQ, D).astype(jnp.float32)

    for r in ag_descs:
        r.wait_send()
    for c in range(N_DEV):

        @pl.when(my != c)
        def _(c=c):
            _rs_desc(pbuf, rs_buf, rs_send, rs_recv, my, c).wait_send()


def kernel(x, Wq, K_ext, V_ext, Wo):
    out = pl.pallas_call(
        _fused_body,
        out_shape=jax.ShapeDtypeStruct((SQ, D), jnp.float32),
        in_specs=[
            pl.BlockSpec(memory_space=pltpu.VMEM),
            pl.BlockSpec(memory_space=pltpu.VMEM),
            pl.BlockSpec(memory_space=pl.ANY),
            pl.BlockSpec(memory_space=pl.ANY),
            pl.BlockSpec(memory_space=pltpu.VMEM),
        ],
        out_specs=pl.BlockSpec(memory_space=pltpu.VMEM),
        scratch_shapes=[
            pltpu.VMEM((HQ_LOCAL, SQ, DH), jnp.float32),
            pltpu.VMEM((HQ_LOCAL, SQ, DH), jnp.float32),
            pltpu.VMEM((SQ, D), jnp.bfloat16),
            pltpu.VMEM((N_DEV, CHUNK, D), jnp.bfloat16),
            pltpu.VMEM((N_DEV, CHUNK, D), jnp.bfloat16),
            pltpu.SemaphoreType.DMA((2 * HQ_LOCAL,)),
            pltpu.SemaphoreType.DMA((N_DEV,)),
            pltpu.SemaphoreType.DMA((N_DEV,)),
            pltpu.SemaphoreType.DMA((N_DEV - 1,)),
            pltpu.SemaphoreType.DMA((N_DEV,)),
        ],
    )(x[0], Wq, K_ext[0], V_ext[0], Wo)

    return out.reshape(1, SQ, D)


# device time: 93103 ns/iter; 1.1496x vs baseline; 1.1496x over previous
import jax
import jax.numpy as jnp
from jax import lax
from jax.experimental import pallas as pl
from jax.experimental.pallas import tpu as pltpu

N_DEV = 32
SQ = 1024
D = 1024
HQ = 256
HQ_LOCAL = 8
DH = 128
BLK = 64
SCALE = 0.08838834764831843
CHUNK = SQ // N_DEV
NBLK = 4
RB = SQ // NBLK
CPB = RB // CHUNK


def _rs_desc(pbuf, rs_buf, rs_send, rs_recv, my, c):
    return pltpu.make_async_remote_copy(
        src_ref=pbuf.at[pl.ds(c * CHUNK, CHUNK), :],
        dst_ref=rs_buf.at[my],
        send_sem=rs_send.at[c],
        recv_sem=rs_recv.at[my],
        device_id=(c,),
        device_id_type=pl.DeviceIdType.MESH,
    )


def _fused_body(x_ref, wq_ref, k_any, v_any, wo_ref, out_ref,
                k_vmem, v_vmem, pbuf, rs_buf, ag_buf,
                kv_sems, rs_send, rs_recv, ag_send, ag_recv):
    my = lax.axis_index("i")
    h0 = my * HQ_LOCAL

    kv_copies = []
    for h in range(HQ_LOCAL):
        ck = pltpu.make_async_copy(
            k_any.at[:, h0 + h, :], k_vmem.at[h], kv_sems.at[h])
        cv = pltpu.make_async_copy(
            v_any.at[:, h0 + h, :], v_vmem.at[h], kv_sems.at[HQ_LOCAL + h])
        ck.start()
        cv.start()
        kv_copies.append(ck)
        kv_copies.append(cv)

    Q = jnp.dot(x_ref[:, :], wq_ref[:, :], preferred_element_type=jnp.float32)
    Wo = wo_ref[:, :]

    for c in kv_copies:
        c.wait()

    for b in range(NBLK):
        r0 = b * RB
        ncol = (b + 1) * RB
        qblk = (r0 + lax.broadcasted_iota(jnp.int32, (RB, ncol), 0)) // BLK
        kblk = lax.broadcasted_iota(jnp.int32, (RB, ncol), 1) // BLK
        mask = kblk <= qblk
        ctx_parts = []
        for h in range(HQ_LOCAL):
            q_h = Q[r0:r0 + RB, h * DH:(h + 1) * DH]
            s = lax.dot_general(
                q_h, k_vmem[h][:ncol, :], (((1,), (1,)), ((), ())),
                preferred_element_type=jnp.float32,
            ) * SCALE
            s = jnp.where(mask, s, -1e9)
            m = jnp.max(s, axis=1, keepdims=True)
            w = jnp.exp(s - m)
            w = w / jnp.sum(w, axis=1, keepdims=True)
            ctx_parts.append(
                jnp.dot(w, v_vmem[h][:ncol, :],
                        preferred_element_type=jnp.float32))
        ctx_b = jnp.concatenate(ctx_parts, axis=1)
        pb = jnp.dot(ctx_b, Wo, preferred_element_type=jnp.float32)
        pbuf[pl.ds(r0, RB), :] = pb.astype(jnp.bfloat16)

        for j in range(CPB):
            c = b * CPB + j

            @pl.when(my != c)
            def _(c=c):
                _rs_desc(pbuf, rs_buf, rs_send, rs_recv, my, c).start()

    pbuf_my = pbuf[pl.ds(my * CHUNK, CHUNK), :]
    rs_buf[pl.ds(my, 1)] = pbuf_my[None, :, :]

    for k in range(1, N_DEV):
        s = lax.rem(my + k, N_DEV)
        pltpu.make_async_remote_copy(
            src_ref=pbuf.at[pl.ds(0, CHUNK), :],
            dst_ref=rs_buf.at[s],
            send_sem=rs_send.at[0],
            recv_sem=rs_recv.at[s],
            device_id=(s,),
            device_id_type=pl.DeviceIdType.MESH,
        ).wait_recv()

    red = jnp.sum(rs_buf[:, :, :].astype(jnp.float32), axis=0)
    ag_buf[pl.ds(my, 1)] = red.astype(jnp.bfloat16)[None, :, :]

    ag_descs = []
    for k in range(1, N_DEV):
        peer = lax.rem(my + k, N_DEV)
        rdma = pltpu.make_async_remote_copy(
            src_ref=ag_buf.at[my],
            dst_ref=ag_buf.at[my],
            send_sem=ag_send.at[k - 1],
            recv_sem=ag_recv.at[my],
            device_id=(peer,),
            device_id_type=pl.DeviceIdType.MESH,
        )
        rdma.start()
        ag_descs.append(rdma)

    for k in range(1, N_DEV):
        s = lax.rem(my + k, N_DEV)
        pltpu.make_async_remote_copy(
            src_ref=ag_buf.at[0],
            dst_ref=ag_buf.at[s],
            send_sem=ag_send.at[0],
            recv_sem=ag_recv.at[s],
            device_id=(s,),
            device_id_type=pl.DeviceIdType.MESH,
        ).wait_recv()

    out_ref[:, :] = ag_buf[:, :, :].reshape(SQ, D).astype(jnp.float32)

    for r in ag_descs:
        r.wait_send()
    for c in range(N_DEV):

        @pl.when(my != c)
        def _(c=c):
            _rs_desc(pbuf, rs_buf, rs_send, rs_recv, my, c).wait_send()


def kernel(x, Wq, K_ext, V_ext, Wo):
    out = pl.pallas_call(
        _fused_body,
        out_shape=jax.ShapeDtypeStruct((SQ, D), jnp.float32),
        in_specs=[
            pl.BlockSpec(memory_space=pltpu.VMEM),
            pl.BlockSpec(memory_space=pltpu.VMEM),
            pl.BlockSpec(memory_space=pl.ANY),
            pl.BlockSpec(memory_space=pl.ANY),
            pl.BlockSpec(memory_space=pltpu.VMEM),
        ],
        out_specs=pl.BlockSpec(memory_space=pltpu.VMEM),
        scratch_shapes=[
            pltpu.VMEM((HQ_LOCAL, SQ, DH), jnp.float32),
            pltpu.VMEM((HQ_LOCAL, SQ, DH), jnp.float32),
            pltpu.VMEM((SQ, D), jnp.bfloat16),
            pltpu.VMEM((N_DEV, CHUNK, D), jnp.bfloat16),
            pltpu.VMEM((N_DEV, CHUNK, D), jnp.bfloat16),
            pltpu.SemaphoreType.DMA((2 * HQ_LOCAL,)),
            pltpu.SemaphoreType.DMA((N_DEV,)),
            pltpu.SemaphoreType.DMA((N_DEV,)),
            pltpu.SemaphoreType.DMA((N_DEV - 1,)),
            pltpu.SemaphoreType.DMA((N_DEV,)),
        ],
    )(x[0], Wq, K_ext[0], V_ext[0], Wo)

    return out.reshape(1, SQ, D)


# device time: 81946 ns/iter; 1.3061x vs baseline; 1.1362x over previous
import jax
import jax.numpy as jnp
from jax import lax
from jax.experimental import pallas as pl
from jax.experimental.pallas import tpu as pltpu

N_DEV = 32
SQ = 1024
D = 1024
HQ = 256
HQ_LOCAL = 8
DH = 128
BLK = 64
SCALE = 0.08838834764831843
NCH = 64
CH = SQ // NCH
NBLK = 4
RB = SQ // NBLK
CPB = RB // CH


def _rs_desc(pbuf, rs_buf, rs_send, rs_recv, my, c):
    slot = 32 * (c // 32) + my
    return pltpu.make_async_remote_copy(
        src_ref=pbuf.at[pl.ds(c * CH, CH), :],
        dst_ref=rs_buf.at[slot],
        send_sem=rs_send.at[c],
        recv_sem=rs_recv.at[slot],
        device_id=(c % 32,),
        device_id_type=pl.DeviceIdType.MESH,
    )


def _fused_body(x_ref, wq_ref, k_any, v_any, wo_ref, out_ref,
                k_vmem, v_vmem, pbuf, rs_buf, ag_buf,
                kv_sems, rs_send, rs_recv, ag_send, ag_recv):
    my = lax.axis_index("i")
    h0 = my * HQ_LOCAL

    kv_copies = []
    for h in range(HQ_LOCAL):
        ck = pltpu.make_async_copy(
            k_any.at[:, h0 + h, :], k_vmem.at[h], kv_sems.at[h])
        cv = pltpu.make_async_copy(
            v_any.at[:, h0 + h, :], v_vmem.at[h], kv_sems.at[HQ_LOCAL + h])
        ck.start()
        cv.start()
        kv_copies.append(ck)
        kv_copies.append(cv)

    Q = jnp.dot(x_ref[:, :], wq_ref[:, :], preferred_element_type=jnp.float32)
    Wo = wo_ref[:, :]

    for c in kv_copies:
        c.wait()

    for b in range(NBLK):
        r0 = b * RB
        ncol = (b + 1) * RB
        qblk = (r0 + lax.broadcasted_iota(jnp.int32, (RB, ncol), 0)) // BLK
        kblk = lax.broadcasted_iota(jnp.int32, (RB, ncol), 1) // BLK
        mask = kblk <= qblk
        ctx_parts = []
        for h in range(HQ_LOCAL):
            q_h = Q[r0:r0 + RB, h * DH:(h + 1) * DH]
            s = lax.dot_general(
                q_h, k_vmem[h][:ncol, :], (((1,), (1,)), ((), ())),
                preferred_element_type=jnp.float32,
            ) * SCALE
            s = jnp.where(mask, s, -1e9)
            m = jnp.max(s, axis=1, keepdims=True)
            w = jnp.exp(s - m)
            w = w / jnp.sum(w, axis=1, keepdims=True)
            ctx_parts.append(
                jnp.dot(w, v_vmem[h][:ncol, :],
                        preferred_element_type=jnp.float32))
        ctx_b = jnp.concatenate(ctx_parts, axis=1)
        pb = jnp.dot(ctx_b, Wo, preferred_element_type=jnp.float32)
        pbuf[pl.ds(r0, RB), :] = pb.astype(jnp.bfloat16)

        for j in range(CPB):
            c = b * CPB + j

            @pl.when(my != c % 32)
            def _(c=c):
                _rs_desc(pbuf, rs_buf, rs_send, rs_recv, my, c).start()

    for half in (0, 1):
        c_own = half * 32
        rs_buf[pl.ds(half * 32 + my, 1)] = (
            pbuf[pl.ds((c_own + my) * CH, CH), :][None, :, :])

    for half in (0, 1):
        for k in range(1, N_DEV):
            s = lax.rem(my + k, N_DEV)
            slot = half * 32 + s
            pltpu.make_async_remote_copy(
                src_ref=pbuf.at[pl.ds(0, CH), :],
                dst_ref=rs_buf.at[slot],
                send_sem=rs_send.at[0],
                recv_sem=rs_recv.at[slot],
                device_id=(s,),
                device_id_type=pl.DeviceIdType.MESH,
            ).wait_recv()

    vals = rs_buf[:, :, :].astype(jnp.float32)
    for half in (0, 1):
        red = jnp.sum(vals[half * 32:(half + 1) * 32], axis=0)
        ag_buf[pl.ds(half * 32 + my, 1)] = red.astype(jnp.bfloat16)[None, :, :]

    ag_descs = []
    for k in range(1, N_DEV):
        peer = lax.rem(my + k, N_DEV)
        for half in (0, 1):
            rdma = pltpu.make_async_remote_copy(
                src_ref=ag_buf.at[half * 32 + my],
                dst_ref=ag_buf.at[half * 32 + my],
                send_sem=ag_send.at[(k - 1) * 2 + half],
                recv_sem=ag_recv.at[half * 32 + my],
                device_id=(peer,),
                device_id_type=pl.DeviceIdType.MESH,
            )
            rdma.start()
            ag_descs.append(rdma)

    for half in (0, 1):
        for k in range(1, N_DEV):
            s = lax.rem(my + k, N_DEV)
            slot = half * 32 + s
            pltpu.make_async_remote_copy(
                src_ref=ag_buf.at[0],
                dst_ref=ag_buf.at[slot],
                send_sem=ag_send.at[0],
                recv_sem=ag_recv.at[slot],
                device_id=(s,),
                device_id_type=pl.DeviceIdType.MESH,
            ).wait_recv()

    out_ref[:, :] = ag_buf[:, :, :].reshape(SQ, D).astype(jnp.float32)

    for r in ag_descs:
        r.wait_send()
    for c in range(NCH):

        @pl.when(my != c % 32)
        def _(c=c):
            _rs_desc(pbuf, rs_buf, rs_send, rs_recv, my, c).wait_send()


def kernel(x, Wq, K_ext, V_ext, Wo):
    out = pl.pallas_call(
        _fused_body,
        out_shape=jax.ShapeDtypeStruct((SQ, D), jnp.float32),
        in_specs=[
            pl.BlockSpec(memory_space=pltpu.VMEM),
            pl.BlockSpec(memory_space=pltpu.VMEM),
            pl.BlockSpec(memory_space=pl.ANY),
            pl.BlockSpec(memory_space=pl.ANY),
            pl.BlockSpec(memory_space=pltpu.VMEM),
        ],
        out_specs=pl.BlockSpec(memory_space=pltpu.VMEM),
        scratch_shapes=[
            pltpu.VMEM((HQ_LOCAL, SQ, DH), jnp.float32),
            pltpu.VMEM((HQ_LOCAL, SQ, DH), jnp.float32),
            pltpu.VMEM((SQ, D), jnp.bfloat16),
            pltpu.VMEM((NCH, CH, D), jnp.bfloat16),
            pltpu.VMEM((NCH, CH, D), jnp.bfloat16),
            pltpu.SemaphoreType.DMA((2 * HQ_LOCAL,)),
            pltpu.SemaphoreType.DMA((NCH,)),
            pltpu.SemaphoreType.DMA((NCH,)),
            pltpu.SemaphoreType.DMA((2 * (N_DEV - 1),)),
            pltpu.SemaphoreType.DMA((NCH,)),
        ],
    )(x[0], Wq, K_ext[0], V_ext[0], Wo)

    return out.reshape(1, SQ, D)
